# baseline (speedup 1.0000x reference)
import functools

import jax
import jax.numpy as jnp
from jax import lax
from jax.experimental import pallas as pl
from jax.experimental.pallas import tpu as pltpu
from jax.experimental.pallas import tpu_sc as plsc


def _transpose_tc(embed_table):
    hw, d = embed_table.shape

    def body(e_ref, o_ref):
        o_ref[...] = e_ref[...].T

    return pl.pallas_call(
        body,
        in_specs=[pl.BlockSpec((hw, d), lambda: (0, 0))],
        out_specs=pl.BlockSpec((d, hw), lambda: (0, 0)),
        out_shape=jax.ShapeDtypeStruct((d, hw), embed_table.dtype),
    )(embed_table)


def kernel(x, embed_table):
    b, _, h, w = x.shape
    hw = h * w
    d = embed_table.shape[1]

    info = plsc.get_sparse_core_info()
    nc, ns = info.num_cores, info.num_subcores
    nw = nc * ns
    dblk = 8
    bpw = b // (nw // (d // dblk))

    mesh = plsc.VectorSubcoreMesh(core_axis_name="c", subcore_axis_name="s")

    @functools.partial(
        pl.kernel,
        out_type=jax.ShapeDtypeStruct((b, d, h, w), jnp.float32),
        mesh=mesh,
        scratch_types=[
            pltpu.VMEM((dblk, h, w), jnp.float32),
            pltpu.SemaphoreType.DMA,
        ],
    )
    def sc_broadcast(t_hbm, out_hbm, rows_v, sem):
        wid = lax.axis_index("s") * nc + lax.axis_index("c")
        db = wid // 2
        par = wid % 2
        pltpu.sync_copy(t_hbm.at[pl.ds(db * dblk, dblk)], rows_v)
        for i in range(bpw):
            pltpu.async_copy(
                rows_v, out_hbm.at[par + 2 * i, pl.ds(db * dblk, dblk)], sem)
        for i in range(bpw):
            pltpu.make_async_copy(
                rows_v, out_hbm.at[par + 2 * i, pl.ds(db * dblk, dblk)],
                sem).wait()

    t_table = _transpose_tc(embed_table).reshape(d, h, w)
    return sc_broadcast(t_table)


# speedup vs baseline: 13.7833x; 13.7833x over previous
import jax
import jax.numpy as jnp
from jax.experimental import pallas as pl


def kernel(x, embed_table):
    b, _, h, w = x.shape
    hw = h * w
    d = embed_table.shape[1]

    bb = 8
    grid = b // bb

    def body(e_ref, o_ref):
        o_ref[...] = jnp.broadcast_to(e_ref[...][None], (bb, hw, d))

    out = pl.pallas_call(
        body,
        grid=(grid,),
        in_specs=[pl.BlockSpec((hw, d), lambda i: (0, 0))],
        out_specs=pl.BlockSpec((bb, hw, d), lambda i: (i, 0, 0)),
        out_shape=jax.ShapeDtypeStruct((b, hw, d), embed_table.dtype),
    )(embed_table)
    return out.reshape(b, h, w, d).transpose(0, 3, 1, 2)
